# baseline (speedup 1.0000x reference)
import jax
import jax.numpy as jnp
from jax import lax
from jax.experimental import pallas as pl
from jax.experimental.pallas import tpu as pltpu
from jax.experimental.pallas import tpu_sc as plsc

_BATCH = 16384
_DIM = 64
_NC = 2
_NS = 16
_NW = _NC * _NS
_ROWS_PER_W = _BATCH // _NW
_CHUNK = 128
_NCHUNK = _ROWS_PER_W // _CHUNK
_L = 16
_GROUPS = _ROWS_PER_W // _L


def _sc_body(idx_u_hbm, idx_i_hbm, table_u_hbm, table_i_hbm, out_hbm,
             idx_u, idx_i, rows_u, rows_i, out_v, sem):
    wid = lax.axis_index("s") * _NC + lax.axis_index("c")
    base = wid * _ROWS_PER_W

    pltpu.sync_copy(idx_u_hbm.at[wid], idx_u)
    pltpu.sync_copy(idx_i_hbm.at[wid], idx_i)

    copies = []
    for j in range(_NCHUNK):
        dst = rows_u.at[pl.ds(j * _CHUNK, _CHUNK), :]
        copies.append(pltpu.async_copy(table_u_hbm.at[idx_u.at[j]], dst, sem))
    for j in range(_NCHUNK):
        dst = rows_i.at[pl.ds(j * _CHUNK, _CHUNK), :]
        copies.append(pltpu.async_copy(table_i_hbm.at[idx_i.at[j]], dst, sem))
    for c in copies:
        c.wait()

    lane = lax.iota(jnp.int32, _L)
    perms = [lane ^ k for k in (1, 2, 4, 8)]
    gd = lax.GatherDimensionNumbers(
        offset_dims=(), collapsed_slice_dims=(0,), start_index_map=(0,))

    def shuffle(x, p):
        return lax.gather(x, p[:, None], gd, slice_sizes=(1,),
                          mode=lax.GatherScatterMode.PROMISE_IN_BOUNDS)

    def group_body(g, carry):
        res = jnp.zeros((_L,), jnp.float32)
        for j in range(_L):
            r = g * _L + j
            acc = jnp.zeros((_L,), jnp.float32)
            for h in range(_DIM // 32):
                uh = plsc.unpack(rows_u[r, pl.ds(h * 32, 32)],
                                 format=plsc.PackFormat.INTERLEAVED)
                ih = plsc.unpack(rows_i[r, pl.ds(h * 32, 32)],
                                 format=plsc.PackFormat.INTERLEAVED)
                acc = acc + uh[0] * ih[0] + uh[1] * ih[1]
            for p in perms:
                acc = acc + shuffle(acc, p)
            res = jnp.where(lane == j, acc, res)
        out_v[pl.ds(g * _L, _L)] = res
        return carry

    lax.fori_loop(0, _GROUPS, group_body, 0)

    pltpu.sync_copy(out_v, out_hbm.at[pl.ds(base, _ROWS_PER_W)])


@jax.jit
def _run(idx_u, idx_i, table_u, table_i):
    mesh = plsc.VectorSubcoreMesh(core_axis_name="c", subcore_axis_name="s")
    f = pl.kernel(
        _sc_body,
        mesh=mesh,
        compiler_params=pltpu.CompilerParams(
            use_tc_tiling_on_sc=False, needs_layout_passes=False),
        out_type=jax.ShapeDtypeStruct((_BATCH,), jnp.float32),
        scratch_types=[
            pltpu.VMEM((_NCHUNK, _CHUNK), jnp.int32),
            pltpu.VMEM((_NCHUNK, _CHUNK), jnp.int32),
            pltpu.VMEM((_ROWS_PER_W, _DIM), jnp.bfloat16),
            pltpu.VMEM((_ROWS_PER_W, _DIM), jnp.bfloat16),
            pltpu.VMEM((_ROWS_PER_W,), jnp.float32),
            pltpu.SemaphoreType.DMA,
        ],
    )
    return f(idx_u, idx_i, table_u, table_i)


def kernel(inputs, user_table, item_table):
    user_idx = inputs[:, 0].reshape(_NW, _NCHUNK, _CHUNK)
    item_idx = inputs[:, 1].reshape(_NW, _NCHUNK, _CHUNK)
    return _run(user_idx, item_idx,
                user_table.astype(jnp.bfloat16),
                item_table.astype(jnp.bfloat16))


# speedup vs baseline: 1.3014x; 1.3014x over previous
import jax
import jax.numpy as jnp
from jax import lax
from jax.experimental import pallas as pl
from jax.experimental.pallas import tpu as pltpu
from jax.experimental.pallas import tpu_sc as plsc

_BATCH = 16384
_DIM = 64
_PAIR = 2 * _DIM
_NC = 2
_NS = 16
_NW = _NC * _NS
_ROWS_PER_W = _BATCH // _NW
_HALF = _ROWS_PER_W // 2
_L = 16
_GROUPS = _HALF // _L


def _sc_body(idx_u_hbm, idx_i_hbm, table_u_hbm, table_i_hbm, out_hbm,
             idx_u, idx_i, rows_u, rows_i, out_v, sem):
    wid = lax.axis_index("s") * _NC + lax.axis_index("c")
    base = wid * _ROWS_PER_W

    pltpu.sync_copy(idx_u_hbm.at[wid], idx_u)
    pltpu.sync_copy(idx_i_hbm.at[wid], idx_i)

    lane = lax.iota(jnp.int32, _L)
    zero = jnp.zeros((_L,), jnp.int32)
    perms = [lane ^ k for k in (1, 2, 4, 8)]
    gd = lax.GatherDimensionNumbers(
        offset_dims=(), collapsed_slice_dims=(0,), start_index_map=(0,))

    def shuffle(x, p):
        return lax.gather(x, p[:, None], gd, slice_sizes=(1,),
                          mode=lax.GatherScatterMode.PROMISE_IN_BOUNDS)

    def extract(v, j):
        return lax.reduce_sum(jnp.where(lane == j, v, zero), axes=(0,))

    for half in range(2):
        hbase = half * _HALF

        def fire(g, carry):
            vu = idx_u[pl.ds(hbase + g * _L, _L)]
            vi = idx_i[pl.ds(hbase + g * _L, _L)]
            for j in range(_L):
                ru = extract(vu, j)
                ri = extract(vi, j)
                r = g * _L + j
                pltpu.async_copy(table_u_hbm.at[ru // 2], rows_u.at[r], sem)
                pltpu.async_copy(table_i_hbm.at[ri // 2], rows_i.at[r], sem)
            return carry

        lax.fori_loop(0, _GROUPS, fire, 0)

        def drain(j, carry):
            pltpu.make_async_copy(table_u_hbm.at[0], rows_u.at[j], sem).wait()
            pltpu.make_async_copy(table_i_hbm.at[0], rows_i.at[j], sem).wait()
            return carry

        lax.fori_loop(0, _HALF, drain, 0)

        def group_body(g, carry):
            vu = idx_u[pl.ds(hbase + g * _L, _L)]
            vi = idx_i[pl.ds(hbase + g * _L, _L)]
            res = jnp.zeros((_L,), jnp.float32)
            for j in range(_L):
                r = g * _L + j
                pu = extract(vu, j) % 2 * _DIM
                pi = extract(vi, j) % 2 * _DIM
                acc = jnp.zeros((_L,), jnp.float32)
                for c in range(_DIM // _L):
                    acc = acc + (rows_u[r, pl.ds(pu + c * _L, _L)] *
                                 rows_i[r, pl.ds(pi + c * _L, _L)])
                for p in perms:
                    acc = acc + shuffle(acc, p)
                res = jnp.where(lane == j, acc, res)
            out_v[pl.ds(hbase + g * _L, _L)] = res
            return carry

        lax.fori_loop(0, _GROUPS, group_body, 0)

    pltpu.sync_copy(out_v, out_hbm.at[pl.ds(base, _ROWS_PER_W)])


@jax.jit
def _run(idx_u, idx_i, table_u, table_i):
    mesh = plsc.VectorSubcoreMesh(core_axis_name="c", subcore_axis_name="s")
    f = pl.kernel(
        _sc_body,
        mesh=mesh,
        compiler_params=pltpu.CompilerParams(needs_layout_passes=False),
        out_type=jax.ShapeDtypeStruct((_BATCH,), jnp.float32),
        scratch_types=[
            pltpu.VMEM((_ROWS_PER_W,), jnp.int32),
            pltpu.VMEM((_ROWS_PER_W,), jnp.int32),
            pltpu.VMEM((_HALF, _PAIR), jnp.float32),
            pltpu.VMEM((_HALF, _PAIR), jnp.float32),
            pltpu.VMEM((_ROWS_PER_W,), jnp.float32),
            pltpu.SemaphoreType.DMA,
        ],
    )
    return f(idx_u, idx_i, table_u, table_i)


def kernel(inputs, user_table, item_table):
    user_idx = inputs[:, 0].reshape(_NW, _ROWS_PER_W)
    item_idx = inputs[:, 1].reshape(_NW, _ROWS_PER_W)
    return _run(user_idx, item_idx,
                user_table.reshape(500000, _PAIR),
                item_table.reshape(500000, _PAIR))


# speedup vs baseline: 2.3845x; 1.8322x over previous
import jax
import jax.numpy as jnp
from jax import lax
from jax.experimental import pallas as pl
from jax.experimental.pallas import tpu as pltpu
from jax.experimental.pallas import tpu_sc as plsc

_BATCH = 16384
_DIM = 64
_NC = 2
_NS = 16
_NW = _NC * _NS
_HPW = _BATCH // _NW
_L = 16
_GROUPS = _HPW // _L
_SW = 128
_META = 528


def _extract(v, sel, lane):
    mask = lane == jnp.full((_L,), 1, jnp.int32) * sel
    return lax.reduce_sum(jnp.where(mask, v, jnp.zeros((_L,), v.dtype)), axes=(0,))


def _chunk_extract(ref, j, lane):
    base = j // _L * _L
    return _extract(ref[pl.ds(base, _L)], j - base, lane)


def _stripe_body(rm_u_hbm, uniq_u_hbm, starts_u_hbm,
                 rm_i_hbm, uniq_i_hbm, starts_i_hbm,
                 tab_u_hbm, tab_i_hbm,
                 inter_u_hbm, inter_i_hbm,
                 rm_v, uniq_v, starts_v, buf, res_v, fsem, wsem):
    wid = lax.axis_index("s") * _NC + lax.axis_index("c")
    base = wid * _HPW
    lane = lax.iota(jnp.int32, _L)

    passes = [(rm_u_hbm, uniq_u_hbm, starts_u_hbm, tab_u_hbm, inter_u_hbm),
              (rm_i_hbm, uniq_i_hbm, starts_i_hbm, tab_i_hbm, inter_i_hbm)]
    for rm_hbm, uniq_hbm, starts_hbm, tab_hbm, inter_hbm in passes:
        pltpu.sync_copy(rm_hbm.at[wid], rm_v)
        pltpu.sync_copy(uniq_hbm.at[wid], uniq_v)
        pltpu.sync_copy(starts_hbm.at[wid], starts_v)

        nuniq = _chunk_extract(starts_v, 513, lane)

        def fetch(k, par):
            s = _chunk_extract(uniq_v, k, lane)
            off = pl.multiple_of(s * _SW, _SW)
            pltpu.async_copy(tab_hbm.at[:, pl.ds(off, _SW)], buf.at[par], fsem)

        fetch(0, 0)
        fetch(1, 1)

        def outer(k, carry):
            pltpu.make_async_copy(
                tab_hbm.at[:, pl.ds(0, _SW)], buf.at[0], fsem).wait()
            par = k % 2
            lo = _chunk_extract(starts_v, k, lane)
            hi = _chunk_extract(starts_v, k + 1, lane)

            def hit(j, c2):
                rm_j = _chunk_extract(rm_v, j, lane)
                for c in range(_DIM // _L):
                    g = plsc.load_gather(
                        buf, [jnp.full((_L,), 1, jnp.int32) * par,
                              c * _L + lane,
                              jnp.full((_L,), 1, jnp.int32) * rm_j])
                    res_v[j, pl.ds(c * _L, _L)] = g
                pltpu.async_copy(res_v.at[j], inter_hbm.at[base + j], wsem)
                return c2

            lax.fori_loop(lo, hi, hit, 0)
            fetch(k + 2, par)
            return carry

        lax.fori_loop(0, nuniq, outer, 0)

        pltpu.make_async_copy(tab_hbm.at[:, pl.ds(0, _SW)], buf.at[0], fsem).wait()
        pltpu.make_async_copy(tab_hbm.at[:, pl.ds(0, _SW)], buf.at[0], fsem).wait()

        def drain(j, carry):
            pltpu.make_async_copy(res_v.at[0], inter_hbm.at[0], wsem).wait()
            return carry

        lax.fori_loop(0, _HPW, drain, 0)


def _dot_body(pos_u_hbm, pos_i_hbm, inter_u_hbm, inter_i_hbm, out_hbm,
              pos_u, pos_i, rows, out_v, sem):
    wid = lax.axis_index("s") * _NC + lax.axis_index("c")
    base = wid * _HPW

    pltpu.sync_copy(pos_u_hbm.at[wid], pos_u)
    pltpu.sync_copy(pos_i_hbm.at[wid], pos_i)

    lane = lax.iota(jnp.int32, _L)
    perms = [lane ^ k for k in (1, 2, 4, 8)]
    gd = lax.GatherDimensionNumbers(
        offset_dims=(), collapsed_slice_dims=(0,), start_index_map=(0,))

    def shuffle(x, p):
        return lax.gather(x, p[:, None], gd, slice_sizes=(1,),
                          mode=lax.GatherScatterMode.PROMISE_IN_BOUNDS)

    def fire(g, carry):
        vu = pos_u[pl.ds(g * _L, _L)]
        vi = pos_i[pl.ds(g * _L, _L)]
        for j in range(_L):
            pu = _extract(vu, j, lane)
            pi = _extract(vi, j, lane)
            r = g * _L + j
            pltpu.async_copy(inter_u_hbm.at[pu], rows.at[r, pl.ds(0, _DIM)], sem)
            pltpu.async_copy(inter_i_hbm.at[pi], rows.at[r, pl.ds(_DIM, _DIM)], sem)
        return carry

    lax.fori_loop(0, _GROUPS, fire, 0)

    def drain(j, carry):
        pltpu.make_async_copy(
            inter_u_hbm.at[0], rows.at[j, pl.ds(0, _DIM)], sem).wait()
        pltpu.make_async_copy(
            inter_i_hbm.at[0], rows.at[j, pl.ds(_DIM, _DIM)], sem).wait()
        return carry

    lax.fori_loop(0, _HPW, drain, 0)

    def group_body(g, carry):
        res = jnp.zeros((_L,), jnp.float32)
        for j in range(_L):
            r = g * _L + j
            acc = rows[r, 0:_L] * rows[r, _DIM:_DIM + _L]
            for c in range(1, _DIM // _L):
                acc = acc + (rows[r, pl.ds(c * _L, _L)] *
                             rows[r, pl.ds(_DIM + c * _L, _L)])
            for p in perms:
                acc = acc + shuffle(acc, p)
            res = jnp.where(lane == j, acc, res)
        out_v[pl.ds(g * _L, _L)] = res
        return carry

    lax.fori_loop(0, _GROUPS, group_body, 0)

    pltpu.sync_copy(out_v, out_hbm.at[pl.ds(base, _HPW)])


@jax.jit
def _run(rm_u, uniq_u, starts_u, rm_i, uniq_i, starts_i,
         pos_u, pos_i, tab_u_t, tab_i_t):
    mesh = plsc.VectorSubcoreMesh(core_axis_name="c", subcore_axis_name="s")
    cp = pltpu.CompilerParams(needs_layout_passes=False)
    k1 = pl.kernel(
        _stripe_body,
        mesh=mesh,
        compiler_params=cp,
        out_type=(jax.ShapeDtypeStruct((_BATCH, _DIM), jnp.float32),
                  jax.ShapeDtypeStruct((_BATCH, _DIM), jnp.float32)),
        scratch_types=[
            pltpu.VMEM((_HPW,), jnp.int32),
            pltpu.VMEM((_META,), jnp.int32),
            pltpu.VMEM((_META,), jnp.int32),
            pltpu.VMEM((2, _DIM, _SW), jnp.float32),
            pltpu.VMEM((_HPW, _DIM), jnp.float32),
            pltpu.SemaphoreType.DMA,
            pltpu.SemaphoreType.DMA,
        ],
    )
    inter_u, inter_i = k1(rm_u, uniq_u, starts_u, rm_i, uniq_i, starts_i,
                          tab_u_t, tab_i_t)
    k2 = pl.kernel(
        _dot_body,
        mesh=mesh,
        compiler_params=cp,
        out_type=jax.ShapeDtypeStruct((_BATCH,), jnp.float32),
        scratch_types=[
            pltpu.VMEM((_HPW,), jnp.int32),
            pltpu.VMEM((_HPW,), jnp.int32),
            pltpu.VMEM((_HPW, 2 * _DIM), jnp.float32),
            pltpu.VMEM((_HPW,), jnp.float32),
            pltpu.SemaphoreType.DMA,
        ],
    )
    return k2(pos_u, pos_i, inter_u, inter_i)


def _prep(idx):
    stripe = idx >> 7
    order = jnp.argsort(stripe).astype(jnp.int32)
    rm = (idx & (_SW - 1))[order]
    s2 = stripe[order].reshape(_NW, _HPW)
    isnew = jnp.concatenate(
        [jnp.ones((_NW, 1), jnp.int32),
         (s2[:, 1:] != s2[:, :-1]).astype(jnp.int32)], axis=1)
    slot = jnp.cumsum(isnew, axis=1) - 1
    ks = jnp.arange(_HPW + 1, dtype=jnp.int32)
    starts = jax.vmap(lambda sl: jnp.searchsorted(sl, ks).astype(jnp.int32))(slot)
    nuniq = slot[:, -1:] + 1
    starts2 = jnp.concatenate(
        [starts[:, :_HPW + 1], nuniq,
         jnp.zeros((_NW, _META - _HPW - 2), jnp.int32)], axis=1)
    uqpos = jnp.minimum(
        jnp.concatenate([starts, jnp.full((_NW, _META - _HPW - 1), _HPW,
                                          jnp.int32)], axis=1)[:, :_META],
        _HPW - 1)
    uniq2 = jnp.take_along_axis(s2, uqpos, axis=1)
    pos = jnp.zeros((_BATCH,), jnp.int32).at[order].set(
        jnp.arange(_BATCH, dtype=jnp.int32), unique_indices=True,
        mode="promise_in_bounds").reshape(_NW, _HPW)
    return rm.reshape(_NW, _HPW), uniq2, starts2, pos


def kernel(inputs, user_table, item_table):
    rm_u, uniq_u, starts_u, pos_u = _prep(inputs[:, 0])
    rm_i, uniq_i, starts_i, pos_i = _prep(inputs[:, 1])
    return _run(rm_u, uniq_u, starts_u, rm_i, uniq_i, starts_i,
                pos_u, pos_i, user_table.T, item_table.T)


# speedup vs baseline: 3.1656x; 1.3276x over previous
import jax
import jax.numpy as jnp
from jax import lax
from jax.experimental import pallas as pl
from jax.experimental.pallas import tpu as pltpu
from jax.experimental.pallas import tpu_sc as plsc

_BATCH = 16384
_DIM = 64
_NC = 2
_NS = 16
_NW = _NC * _NS
_HPW = _BATCH // _NW
_L = 16
_GROUPS = _HPW // _L
_SW = 128
_META = 528
_NBUF = 4


def _extract(v, sel, lane):
    mask = lane == jnp.full((_L,), 1, jnp.int32) * sel
    return lax.reduce_sum(jnp.where(mask, v, jnp.zeros((_L,), v.dtype)), axes=(0,))


def _chunk_extract(ref, j, lane):
    base = j // _L * _L
    return _extract(ref[pl.ds(base, _L)], j - base, lane)


def _stripe_body(rm_u_hbm, uniq_u_hbm, starts_u_hbm,
                 rm_i_hbm, uniq_i_hbm, starts_i_hbm,
                 tab_u_hbm, tab_i_hbm,
                 inter_u_hbm, inter_i_hbm,
                 rm_v, uniq_v, starts_v, buf, res_v, fsem, wsem):
    wid = lax.axis_index("s") * _NC + lax.axis_index("c")
    base = wid * _HPW
    lane = lax.iota(jnp.int32, _L)

    passes = [(rm_u_hbm, uniq_u_hbm, starts_u_hbm, tab_u_hbm, inter_u_hbm),
              (rm_i_hbm, uniq_i_hbm, starts_i_hbm, tab_i_hbm, inter_i_hbm)]
    for rm_hbm, uniq_hbm, starts_hbm, tab_hbm, inter_hbm in passes:
        pltpu.sync_copy(rm_hbm.at[wid], rm_v)
        pltpu.sync_copy(uniq_hbm.at[wid], uniq_v)
        pltpu.sync_copy(starts_hbm.at[wid], starts_v)

        nuniq = _chunk_extract(starts_v, 513, lane)

        def fetch(k, par):
            s = _chunk_extract(uniq_v, k, lane)
            off = pl.multiple_of(s * _SW, _SW)
            pltpu.async_copy(tab_hbm.at[:, pl.ds(off, _SW)], buf.at[par], fsem)

        for q in range(_NBUF):
            fetch(q, q)

        def outer(k, carry):
            pltpu.make_async_copy(
                tab_hbm.at[:, pl.ds(0, _SW)], buf.at[0], fsem).wait()
            par = k % _NBUF
            lo = _chunk_extract(starts_v, k, lane)
            hi = _chunk_extract(starts_v, k + 1, lane)

            def hit(j, c2):
                rm_j = _chunk_extract(rm_v, j, lane)
                for c in range(_DIM // _L):
                    g = plsc.load_gather(
                        buf, [jnp.full((_L,), 1, jnp.int32) * par,
                              c * _L + lane,
                              jnp.full((_L,), 1, jnp.int32) * rm_j])
                    res_v[j, pl.ds(c * _L, _L)] = g
                pltpu.async_copy(res_v.at[j], inter_hbm.at[base + j], wsem)
                return c2

            lax.fori_loop(lo, hi, hit, 0)
            fetch(k + _NBUF, par)
            return carry

        lax.fori_loop(0, nuniq, outer, 0)

        for q in range(_NBUF):
            pltpu.make_async_copy(
                tab_hbm.at[:, pl.ds(0, _SW)], buf.at[0], fsem).wait()

        def drain(j, carry):
            pltpu.make_async_copy(res_v.at[0], inter_hbm.at[0], wsem).wait()
            return carry

        lax.fori_loop(0, _HPW, drain, 0)


def _dot_body(pos_u_hbm, pos_i_hbm, inter_u_hbm, inter_i_hbm, out_hbm,
              pos_u, pos_i, rows, out_v, sem):
    wid = lax.axis_index("s") * _NC + lax.axis_index("c")
    base = wid * _HPW

    pltpu.sync_copy(pos_u_hbm.at[wid], pos_u)
    pltpu.sync_copy(pos_i_hbm.at[wid], pos_i)

    lane = lax.iota(jnp.int32, _L)
    perms = [lane ^ k for k in (1, 2, 4, 8)]
    gd = lax.GatherDimensionNumbers(
        offset_dims=(), collapsed_slice_dims=(0,), start_index_map=(0,))

    def shuffle(x, p):
        return lax.gather(x, p[:, None], gd, slice_sizes=(1,),
                          mode=lax.GatherScatterMode.PROMISE_IN_BOUNDS)

    def fire(g, carry):
        vu = pos_u[pl.ds(g * _L, _L)]
        vi = pos_i[pl.ds(g * _L, _L)]
        for j in range(_L):
            pu = _extract(vu, j, lane)
            pi = _extract(vi, j, lane)
            r = g * _L + j
            pltpu.async_copy(inter_u_hbm.at[pu], rows.at[r, pl.ds(0, _DIM)], sem)
            pltpu.async_copy(inter_i_hbm.at[pi], rows.at[r, pl.ds(_DIM, _DIM)], sem)
        return carry

    lax.fori_loop(0, _GROUPS, fire, 0)

    def drain(j, carry):
        pltpu.make_async_copy(
            inter_u_hbm.at[0], rows.at[j, pl.ds(0, _DIM)], sem).wait()
        pltpu.make_async_copy(
            inter_i_hbm.at[0], rows.at[j, pl.ds(_DIM, _DIM)], sem).wait()
        return carry

    lax.fori_loop(0, _HPW, drain, 0)

    def group_body(g, carry):
        res = jnp.zeros((_L,), jnp.float32)
        for j in range(_L):
            r = g * _L + j
            acc = rows[r, 0:_L] * rows[r, _DIM:_DIM + _L]
            for c in range(1, _DIM // _L):
                acc = acc + (rows[r, pl.ds(c * _L, _L)] *
                             rows[r, pl.ds(_DIM + c * _L, _L)])
            for p in perms:
                acc = acc + shuffle(acc, p)
            res = jnp.where(lane == j, acc, res)
        out_v[pl.ds(g * _L, _L)] = res
        return carry

    lax.fori_loop(0, _GROUPS, group_body, 0)

    pltpu.sync_copy(out_v, out_hbm.at[pl.ds(base, _HPW)])


@jax.jit
def _run(rm_u, uniq_u, starts_u, rm_i, uniq_i, starts_i,
         pos_u, pos_i, tab_u_t, tab_i_t):
    mesh = plsc.VectorSubcoreMesh(core_axis_name="c", subcore_axis_name="s")
    cp = pltpu.CompilerParams(needs_layout_passes=False)
    k1 = pl.kernel(
        _stripe_body,
        mesh=mesh,
        compiler_params=cp,
        out_type=(jax.ShapeDtypeStruct((_BATCH, _DIM), jnp.float32),
                  jax.ShapeDtypeStruct((_BATCH, _DIM), jnp.float32)),
        scratch_types=[
            pltpu.VMEM((_HPW,), jnp.int32),
            pltpu.VMEM((_META,), jnp.int32),
            pltpu.VMEM((_META,), jnp.int32),
            pltpu.VMEM((_NBUF, _DIM, _SW), jnp.float32),
            pltpu.VMEM((_HPW, _DIM), jnp.float32),
            pltpu.SemaphoreType.DMA,
            pltpu.SemaphoreType.DMA,
        ],
    )
    inter_u, inter_i = k1(rm_u, uniq_u, starts_u, rm_i, uniq_i, starts_i,
                          tab_u_t, tab_i_t)
    k2 = pl.kernel(
        _dot_body,
        mesh=mesh,
        compiler_params=cp,
        out_type=jax.ShapeDtypeStruct((_BATCH,), jnp.float32),
        scratch_types=[
            pltpu.VMEM((_HPW,), jnp.int32),
            pltpu.VMEM((_HPW,), jnp.int32),
            pltpu.VMEM((_HPW, 2 * _DIM), jnp.float32),
            pltpu.VMEM((_HPW,), jnp.float32),
            pltpu.SemaphoreType.DMA,
        ],
    )
    return k2(pos_u, pos_i, inter_u, inter_i)


def _prep(idx2):
    order = jnp.argsort(idx2, axis=-1).astype(jnp.int32)
    idx_s = jnp.take_along_axis(idx2, order, axis=-1)
    s2 = (idx_s >> 7).reshape(2, _NW, _HPW)
    rm2 = (idx_s & (_SW - 1)).reshape(2, _NW, _HPW)
    isnew = jnp.concatenate(
        [jnp.ones((2, _NW, 1), jnp.int32),
         (s2[:, :, 1:] != s2[:, :, :-1]).astype(jnp.int32)], axis=2)
    slot = jnp.cumsum(isnew, axis=2) - 1
    ks = jnp.arange(_HPW + 1, dtype=jnp.int32)
    starts = jax.vmap(jax.vmap(
        lambda sl: jnp.searchsorted(sl, ks).astype(jnp.int32)))(slot)
    nuniq = slot[:, :, -1:] + 1
    starts2 = jnp.concatenate(
        [starts[:, :, :_HPW + 1], nuniq,
         jnp.zeros((2, _NW, _META - _HPW - 2), jnp.int32)], axis=2)
    uqpos = jnp.minimum(
        jnp.concatenate([starts, jnp.full((2, _NW, _META - _HPW - 1), _HPW,
                                          jnp.int32)], axis=2)[:, :, :_META],
        _HPW - 1)
    uniq2 = jnp.take_along_axis(s2, uqpos, axis=2)
    rows2 = jnp.broadcast_to(jnp.arange(2, dtype=jnp.int32)[:, None],
                             (2, _BATCH))
    pos = jnp.zeros((2, _BATCH), jnp.int32).at[rows2, order].set(
        jnp.broadcast_to(jnp.arange(_BATCH, dtype=jnp.int32)[None], (2, _BATCH)),
        unique_indices=True,
        mode="promise_in_bounds").reshape(2, _NW, _HPW)
    return rm2, uniq2, starts2, pos


def kernel(inputs, user_table, item_table):
    rm2, uniq2, starts2, pos = _prep(inputs.T)
    return _run(rm2[0], uniq2[0], starts2[0], rm2[1], uniq2[1], starts2[1],
                pos[0], pos[1], user_table.T, item_table.T)


# speedup vs baseline: 3.3232x; 1.0498x over previous
import jax
import jax.numpy as jnp
from jax import lax
from jax.experimental import pallas as pl
from jax.experimental.pallas import tpu as pltpu
from jax.experimental.pallas import tpu_sc as plsc

_BATCH = 16384
_DIM = 64
_NC = 2
_NS = 16
_NW = _NC * _NS
_HPW = _BATCH // _NW
_L = 16
_GROUPS = _HPW // _L
_SW = 128
_META = 528
_NBUF = 6


def _extract(v, sel, lane):
    mask = lane == jnp.full((_L,), 1, jnp.int32) * sel
    return lax.reduce_sum(jnp.where(mask, v, jnp.zeros((_L,), v.dtype)), axes=(0,))


def _chunk_extract(ref, j, lane):
    base = j // _L * _L
    return _extract(ref[pl.ds(base, _L)], j - base, lane)


def _stripe_body(rm_u_hbm, uniq_u_hbm, starts_u_hbm,
                 rm_i_hbm, uniq_i_hbm, starts_i_hbm,
                 tab_u_hbm, tab_i_hbm,
                 inter_u_hbm, inter_i_hbm,
                 rm_v, uniq_v, starts_v, buf, res_v, fsem, wsem):
    wid = lax.axis_index("s") * _NC + lax.axis_index("c")
    base = wid * _HPW
    lane = lax.iota(jnp.int32, _L)

    passes = [(rm_u_hbm, uniq_u_hbm, starts_u_hbm, tab_u_hbm, inter_u_hbm),
              (rm_i_hbm, uniq_i_hbm, starts_i_hbm, tab_i_hbm, inter_i_hbm)]
    for rm_hbm, uniq_hbm, starts_hbm, tab_hbm, inter_hbm in passes:
        pltpu.sync_copy(rm_hbm.at[wid], rm_v)
        pltpu.sync_copy(uniq_hbm.at[wid], uniq_v)
        pltpu.sync_copy(starts_hbm.at[wid], starts_v)

        nuniq = _chunk_extract(starts_v, 513, lane)

        def fetch(k, par):
            s = _chunk_extract(uniq_v, k, lane)
            off = pl.multiple_of(s * _SW, _SW)
            pltpu.async_copy(tab_hbm.at[:, pl.ds(off, _SW)], buf.at[par], fsem)

        for q in range(_NBUF):
            fetch(q, q)

        def outer(k, carry):
            pltpu.make_async_copy(
                tab_hbm.at[:, pl.ds(0, _SW)], buf.at[0], fsem).wait()
            par = k % _NBUF
            lo = _chunk_extract(starts_v, k, lane)
            hi = _chunk_extract(starts_v, k + 1, lane)

            def hit(j, c2):
                rm_j = _chunk_extract(rm_v, j, lane)
                for c in range(_DIM // _L):
                    g = plsc.load_gather(
                        buf, [jnp.full((_L,), 1, jnp.int32) * par,
                              c * _L + lane,
                              jnp.full((_L,), 1, jnp.int32) * rm_j])
                    res_v[j, pl.ds(c * _L, _L)] = g
                pltpu.async_copy(res_v.at[j], inter_hbm.at[base + j], wsem)
                return c2

            lax.fori_loop(lo, hi, hit, 0)
            fetch(k + _NBUF, par)
            return carry

        lax.fori_loop(0, nuniq, outer, 0)

        for q in range(_NBUF):
            pltpu.make_async_copy(
                tab_hbm.at[:, pl.ds(0, _SW)], buf.at[0], fsem).wait()

        def drain(j, carry):
            pltpu.make_async_copy(res_v.at[0], inter_hbm.at[0], wsem).wait()
            return carry

        lax.fori_loop(0, _HPW, drain, 0)


def _dot_body(pos_u_hbm, pos_i_hbm, inter_u_hbm, inter_i_hbm, out_hbm,
              pos_u, pos_i, rows, out_v, sem):
    wid = lax.axis_index("s") * _NC + lax.axis_index("c")
    base = wid * _HPW

    pltpu.sync_copy(pos_u_hbm.at[wid], pos_u)
    pltpu.sync_copy(pos_i_hbm.at[wid], pos_i)

    lane = lax.iota(jnp.int32, _L)
    perms = [lane ^ k for k in (1, 2, 4, 8)]
    gd = lax.GatherDimensionNumbers(
        offset_dims=(), collapsed_slice_dims=(0,), start_index_map=(0,))

    def shuffle(x, p):
        return lax.gather(x, p[:, None], gd, slice_sizes=(1,),
                          mode=lax.GatherScatterMode.PROMISE_IN_BOUNDS)

    def fire(g, carry):
        vu = pos_u[pl.ds(g * _L, _L)]
        vi = pos_i[pl.ds(g * _L, _L)]
        for j in range(_L):
            pu = _extract(vu, j, lane)
            pi = _extract(vi, j, lane)
            r = g * _L + j
            pltpu.async_copy(inter_u_hbm.at[pu], rows.at[r, pl.ds(0, _DIM)], sem)
            pltpu.async_copy(inter_i_hbm.at[pi], rows.at[r, pl.ds(_DIM, _DIM)], sem)
        return carry

    lax.fori_loop(0, _GROUPS, fire, 0)

    def drain(j, carry):
        pltpu.make_async_copy(
            inter_u_hbm.at[0], rows.at[j, pl.ds(0, _DIM)], sem).wait()
        pltpu.make_async_copy(
            inter_i_hbm.at[0], rows.at[j, pl.ds(_DIM, _DIM)], sem).wait()
        return carry

    lax.fori_loop(0, _HPW, drain, 0)

    def group_body(g, carry):
        res = jnp.zeros((_L,), jnp.float32)
        for j in range(_L):
            r = g * _L + j
            acc = rows[r, 0:_L] * rows[r, _DIM:_DIM + _L]
            for c in range(1, _DIM // _L):
                acc = acc + (rows[r, pl.ds(c * _L, _L)] *
                             rows[r, pl.ds(_DIM + c * _L, _L)])
            for p in perms:
                acc = acc + shuffle(acc, p)
            res = jnp.where(lane == j, acc, res)
        out_v[pl.ds(g * _L, _L)] = res
        return carry

    lax.fori_loop(0, _GROUPS, group_body, 0)

    pltpu.sync_copy(out_v, out_hbm.at[pl.ds(base, _HPW)])


@jax.jit
def _run(rm_u, uniq_u, starts_u, rm_i, uniq_i, starts_i,
         pos_u, pos_i, tab_u_t, tab_i_t):
    mesh = plsc.VectorSubcoreMesh(core_axis_name="c", subcore_axis_name="s")
    cp = pltpu.CompilerParams(needs_layout_passes=False)
    k1 = pl.kernel(
        _stripe_body,
        mesh=mesh,
        compiler_params=cp,
        out_type=(jax.ShapeDtypeStruct((_BATCH, _DIM), jnp.float32),
                  jax.ShapeDtypeStruct((_BATCH, _DIM), jnp.float32)),
        scratch_types=[
            pltpu.VMEM((_HPW,), jnp.int32),
            pltpu.VMEM((_META,), jnp.int32),
            pltpu.VMEM((_META,), jnp.int32),
            pltpu.VMEM((_NBUF, _DIM, _SW), jnp.float32),
            pltpu.VMEM((_HPW, _DIM), jnp.float32),
            pltpu.SemaphoreType.DMA,
            pltpu.SemaphoreType.DMA,
        ],
    )
    inter_u, inter_i = k1(rm_u, uniq_u, starts_u, rm_i, uniq_i, starts_i,
                          tab_u_t, tab_i_t)
    k2 = pl.kernel(
        _dot_body,
        mesh=mesh,
        compiler_params=cp,
        out_type=jax.ShapeDtypeStruct((_BATCH,), jnp.float32),
        scratch_types=[
            pltpu.VMEM((_HPW,), jnp.int32),
            pltpu.VMEM((_HPW,), jnp.int32),
            pltpu.VMEM((_HPW, 2 * _DIM), jnp.float32),
            pltpu.VMEM((_HPW,), jnp.float32),
            pltpu.SemaphoreType.DMA,
        ],
    )
    return k2(pos_u, pos_i, inter_u, inter_i)


def _prep(idx2):
    iota2 = jnp.broadcast_to(jnp.arange(_BATCH, dtype=jnp.int32)[None],
                             (2, _BATCH))
    idx_s, order = lax.sort((idx2, iota2), dimension=-1, num_keys=1)
    s2 = (idx_s >> 7).reshape(2, _NW, _HPW)
    rm2 = (idx_s & (_SW - 1)).reshape(2, _NW, _HPW)
    isnew = jnp.concatenate(
        [jnp.ones((2, _NW, 1), jnp.int32),
         (s2[:, :, 1:] != s2[:, :, :-1]).astype(jnp.int32)], axis=2)
    slot = jnp.cumsum(isnew, axis=2) - 1
    ks = jnp.arange(_HPW + 1, dtype=jnp.int32)
    starts = jax.vmap(jax.vmap(
        lambda sl: jnp.searchsorted(sl, ks).astype(jnp.int32)))(slot)
    nuniq = slot[:, :, -1:] + 1
    starts2 = jnp.concatenate(
        [starts[:, :, :_HPW + 1], nuniq,
         jnp.zeros((2, _NW, _META - _HPW - 2), jnp.int32)], axis=2)
    uqpos = jnp.minimum(
        jnp.concatenate([starts, jnp.full((2, _NW, _META - _HPW - 1), _HPW,
                                          jnp.int32)], axis=2)[:, :, :_META],
        _HPW - 1)
    uniq2 = jnp.take_along_axis(s2, uqpos, axis=2)
    rows2 = jnp.broadcast_to(jnp.arange(2, dtype=jnp.int32)[:, None],
                             (2, _BATCH))
    pos = jnp.zeros((2, _BATCH), jnp.int32).at[rows2, order].set(
        jnp.broadcast_to(jnp.arange(_BATCH, dtype=jnp.int32)[None], (2, _BATCH)),
        unique_indices=True,
        mode="promise_in_bounds").reshape(2, _NW, _HPW)
    return rm2, uniq2, starts2, pos


def kernel(inputs, user_table, item_table):
    rm2, uniq2, starts2, pos = _prep(inputs.T)
    return _run(rm2[0], uniq2[0], starts2[0], rm2[1], uniq2[1], starts2[1],
                pos[0], pos[1], user_table.T, item_table.T)


# speedup vs baseline: 3.3442x; 1.0063x over previous
import jax
import jax.numpy as jnp
from jax import lax
from jax.experimental import pallas as pl
from jax.experimental.pallas import tpu as pltpu
from jax.experimental.pallas import tpu_sc as plsc

_BATCH = 16384
_DIM = 64
_NC = 2
_NS = 16
_NW = _NC * _NS
_HPW = _BATCH // _NW
_L = 16
_GROUPS = _HPW // _L
_SW = 128
_META = 528
_NBUF = 6


def _extract(v, sel, lane):
    mask = lane == jnp.full((_L,), 1, jnp.int32) * sel
    return lax.reduce_sum(jnp.where(mask, v, jnp.zeros((_L,), v.dtype)), axes=(0,))


def _chunk_extract(ref, j, lane):
    base = j // _L * _L
    return _extract(ref[pl.ds(base, _L)], j - base, lane)


def _stripe_body(rm_u_hbm, uniq_u_hbm, starts_u_hbm, b_u_hbm,
                 rm_i_hbm, uniq_i_hbm, starts_i_hbm, b_i_hbm,
                 tab_u_hbm, tab_i_hbm,
                 inter_u_hbm, inter_i_hbm,
                 rm_v, uniq_v, starts_v, b_v, buf, res_v, fsem, wsem):
    wid = lax.axis_index("s") * _NC + lax.axis_index("c")
    base = wid * _HPW
    lane = lax.iota(jnp.int32, _L)

    passes = [(rm_u_hbm, uniq_u_hbm, starts_u_hbm, b_u_hbm, tab_u_hbm,
               inter_u_hbm),
              (rm_i_hbm, uniq_i_hbm, starts_i_hbm, b_i_hbm, tab_i_hbm,
               inter_i_hbm)]
    for rm_hbm, uniq_hbm, starts_hbm, b_hbm, tab_hbm, inter_hbm in passes:
        pltpu.sync_copy(rm_hbm.at[wid], rm_v)
        pltpu.sync_copy(uniq_hbm.at[wid], uniq_v)
        pltpu.sync_copy(starts_hbm.at[wid], starts_v)
        pltpu.sync_copy(b_hbm.at[wid], b_v)

        nuniq = _chunk_extract(starts_v, 513, lane)

        def fetch(k, par):
            s = _chunk_extract(uniq_v, k, lane)
            off = pl.multiple_of(s * _SW, _SW)
            pltpu.async_copy(tab_hbm.at[:, pl.ds(off, _SW)], buf.at[par], fsem)

        for q in range(_NBUF):
            fetch(q, q)

        def outer(k, carry):
            pltpu.make_async_copy(
                tab_hbm.at[:, pl.ds(0, _SW)], buf.at[0], fsem).wait()
            par = k % _NBUF
            lo = _chunk_extract(starts_v, k, lane)
            hi = _chunk_extract(starts_v, k + 1, lane)

            def hit(j, c2):
                rm_j = _chunk_extract(rm_v, j, lane)
                for c in range(_DIM // _L):
                    g = plsc.load_gather(
                        buf, [jnp.full((_L,), 1, jnp.int32) * par,
                              c * _L + lane,
                              jnp.full((_L,), 1, jnp.int32) * rm_j])
                    res_v[j, pl.ds(c * _L, _L)] = g
                b_j = _chunk_extract(b_v, j, lane)
                pltpu.async_copy(res_v.at[j], inter_hbm.at[b_j], wsem)
                return c2

            lax.fori_loop(lo, hi, hit, 0)
            fetch(k + _NBUF, par)
            return carry

        lax.fori_loop(0, nuniq, outer, 0)

        for q in range(_NBUF):
            pltpu.make_async_copy(
                tab_hbm.at[:, pl.ds(0, _SW)], buf.at[0], fsem).wait()

        def drain(j, carry):
            pltpu.make_async_copy(res_v.at[0], inter_hbm.at[0], wsem).wait()
            return carry

        lax.fori_loop(0, _HPW, drain, 0)


def _dot_body(inter_u_hbm, inter_i_hbm, out_hbm, rows, out_v, sem):
    wid = lax.axis_index("s") * _NC + lax.axis_index("c")
    base = wid * _HPW

    lane = lax.iota(jnp.int32, _L)
    perms = [lane ^ k for k in (1, 2, 4, 8)]
    gd = lax.GatherDimensionNumbers(
        offset_dims=(), collapsed_slice_dims=(0,), start_index_map=(0,))

    def shuffle(x, p):
        return lax.gather(x, p[:, None], gd, slice_sizes=(1,),
                          mode=lax.GatherScatterMode.PROMISE_IN_BOUNDS)

    def fire(r, carry):
        pltpu.async_copy(inter_u_hbm.at[base + r], rows.at[r, pl.ds(0, _DIM)], sem)
        pltpu.async_copy(inter_i_hbm.at[base + r], rows.at[r, pl.ds(_DIM, _DIM)], sem)
        return carry

    lax.fori_loop(0, _HPW, fire, 0)

    def drain(j, carry):
        pltpu.make_async_copy(
            inter_u_hbm.at[0], rows.at[j, pl.ds(0, _DIM)], sem).wait()
        pltpu.make_async_copy(
            inter_i_hbm.at[0], rows.at[j, pl.ds(_DIM, _DIM)], sem).wait()
        return carry

    lax.fori_loop(0, _HPW, drain, 0)

    def group_body(g, carry):
        res = jnp.zeros((_L,), jnp.float32)
        for j in range(_L):
            r = g * _L + j
            acc = rows[r, 0:_L] * rows[r, _DIM:_DIM + _L]
            for c in range(1, _DIM // _L):
                acc = acc + (rows[r, pl.ds(c * _L, _L)] *
                             rows[r, pl.ds(_DIM + c * _L, _L)])
            for p in perms:
                acc = acc + shuffle(acc, p)
            res = jnp.where(lane == j, acc, res)
        out_v[pl.ds(g * _L, _L)] = res
        return carry

    lax.fori_loop(0, _GROUPS, group_body, 0)

    pltpu.sync_copy(out_v, out_hbm.at[pl.ds(base, _HPW)])


@jax.jit
def _run(rm_u, uniq_u, starts_u, b_u, rm_i, uniq_i, starts_i, b_i,
         tab_u_t, tab_i_t):
    mesh = plsc.VectorSubcoreMesh(core_axis_name="c", subcore_axis_name="s")
    cp = pltpu.CompilerParams(needs_layout_passes=False)
    k1 = pl.kernel(
        _stripe_body,
        mesh=mesh,
        compiler_params=cp,
        out_type=(jax.ShapeDtypeStruct((_BATCH, _DIM), jnp.float32),
                  jax.ShapeDtypeStruct((_BATCH, _DIM), jnp.float32)),
        scratch_types=[
            pltpu.VMEM((_HPW,), jnp.int32),
            pltpu.VMEM((_META,), jnp.int32),
            pltpu.VMEM((_META,), jnp.int32),
            pltpu.VMEM((_HPW,), jnp.int32),
            pltpu.VMEM((_NBUF, _DIM, _SW), jnp.float32),
            pltpu.VMEM((_HPW, _DIM), jnp.float32),
            pltpu.SemaphoreType.DMA,
            pltpu.SemaphoreType.DMA,
        ],
    )
    inter_u, inter_i = k1(rm_u, uniq_u, starts_u, b_u,
                          rm_i, uniq_i, starts_i, b_i,
                          tab_u_t, tab_i_t)
    k2 = pl.kernel(
        _dot_body,
        mesh=mesh,
        compiler_params=cp,
        out_type=jax.ShapeDtypeStruct((_BATCH,), jnp.float32),
        scratch_types=[
            pltpu.VMEM((_HPW, 2 * _DIM), jnp.float32),
            pltpu.VMEM((_HPW,), jnp.float32),
            pltpu.SemaphoreType.DMA,
        ],
    )
    return k2(inter_u, inter_i)


def _prep(idx2):
    iota2 = jnp.broadcast_to(jnp.arange(_BATCH, dtype=jnp.int32)[None],
                             (2, _BATCH))
    idx_s, order = lax.sort((idx2, iota2), dimension=-1, num_keys=1)
    s2 = (idx_s >> 7).reshape(2, _NW, _HPW)
    rm2 = (idx_s & (_SW - 1)).reshape(2, _NW, _HPW)
    isnew = jnp.concatenate(
        [jnp.ones((2, _NW, 1), jnp.int32),
         (s2[:, :, 1:] != s2[:, :, :-1]).astype(jnp.int32)], axis=2)
    slot = jnp.cumsum(isnew, axis=2) - 1
    ks = jnp.arange(_HPW + 1, dtype=jnp.int32)
    starts = jax.vmap(jax.vmap(
        lambda sl: jnp.searchsorted(sl, ks).astype(jnp.int32)))(slot)
    nuniq = slot[:, :, -1:] + 1
    starts2 = jnp.concatenate(
        [starts[:, :, :_HPW + 1], nuniq,
         jnp.zeros((2, _NW, _META - _HPW - 2), jnp.int32)], axis=2)
    uqpos = jnp.minimum(
        jnp.concatenate([starts, jnp.full((2, _NW, _META - _HPW - 1), _HPW,
                                          jnp.int32)], axis=2)[:, :, :_META],
        _HPW - 1)
    uniq2 = jnp.take_along_axis(s2, uqpos, axis=2)
    b2 = order.reshape(2, _NW, _HPW)
    return rm2, uniq2, starts2, b2


def kernel(inputs, user_table, item_table):
    rm2, uniq2, starts2, b2 = _prep(inputs.T)
    return _run(rm2[0], uniq2[0], starts2[0], b2[0],
                rm2[1], uniq2[1], starts2[1], b2[1],
                user_table.T, item_table.T)
